# baseline (speedup 1.0000x reference)
import functools
import math

import jax
import jax.numpy as jnp
from jax import lax
from jax.experimental import pallas as pl
from jax.experimental.pallas import tpu as pltpu

AM_MARGIN = 0.2
AM_SCALE = 30.0
PROTO_W = 10.0
PROTO_B = -5.0
MTL_WEIGHT = 0.6

_SM = AM_SCALE * AM_MARGIN
_EXP_NEG_SM = math.exp(-_SM)
_LOG2E = 1.4426950408889634
_K_AM = AM_SCALE * _LOG2E
_K_PR = PROTO_W * _LOG2E
_NEG = -1e30


def _l2n_bf16(v):
    s = jnp.sum(v * v, axis=-1, keepdims=True)
    return (v * lax.rsqrt(jnp.maximum(s, 1e-24))).astype(jnp.bfloat16)


def _colnorm_bf16(w):
    inv = lax.rsqrt(jnp.maximum(jnp.sum(w * w, axis=0, keepdims=True), 1e-24))
    return (w * inv).astype(jnp.bfloat16)


def _main_kernel(x_ref, labh_ref,
                 w_am_hbm, w_px_hbm,
                 out_ref,
                 wamf, wpxf, wn_am, wn_px, pn_ref, an_ref,
                 sems_am, sem_px,
                 *, n, b, m_utts, hb, hc, ch, kc_n):
    f32 = jnp.float32
    s = pl.program_id(0)
    col0 = pl.multiple_of(s * hc, 256)
    kw = hc // kc_n

    cps_am = []
    for kc in range(kc_n):
        cp = pltpu.make_async_copy(
            w_am_hbm.at[:, pl.ds(pl.multiple_of(col0 + kc * kw, 128), kw)],
            wamf.at[:, kc * kw:(kc + 1) * kw], sems_am.at[kc])
        cp.start()
        cps_am.append(cp)
    cp_px = pltpu.make_async_copy(w_px_hbm.at[:, pl.ds(col0, hc)], wpxf,
                                  sem_px)
    cp_px.start()

    pn_ref[...] = _l2n_bf16(x_ref[:, 0, :])
    arow0 = pl.multiple_of(s * hb, 8)
    anc = x_ref[pl.ds(arow0, hb), 1, :]
    for m in range(2, m_utts):
        anc = anc + x_ref[pl.ds(arow0, hb), m, :]
    if m_utts > 2:
        anc = anc * (1.0 / float(m_utts - 1))
    an_ref[...] = _l2n_bf16(anc)

    pairc = lax.dot_general(pn_ref[...], an_ref[...], (((1,), (1,)), ((), ())),
                            preferred_element_type=f32)
    ri = lax.broadcasted_iota(jnp.int32, (b, hb), 0)
    ci = lax.broadcasted_iota(jnp.int32, (b, hb), 1) + s * hb
    pmask = ri == ci
    out_ref[0, 0:b, 3:4] = jnp.sum(jnp.exp2(_K_PR * pairc), axis=-1,
                                   keepdims=True)
    out_ref[0, 0:b, 4:5] = jnp.sum(jnp.where(pmask, pairc, 0.0), axis=-1,
                                   keepdims=True)

    spc = ch // m_utts
    for t in range(n // ch):
        xn = _l2n_bf16(x_ref[t * spc:(t + 1) * spc, :, :].reshape(ch, -1))
        lloc = (labh_ref[t * spc:(t + 1) * spc, :] - col0).reshape(spc, 1, 1)
        se = tc = mxc = None
        for kc in range(kc_n):
            if t == 0:
                cps_am[kc].wait()
                wn_am[:, kc * kw:(kc + 1) * kw] = _colnorm_bf16(
                    wamf[:, kc * kw:(kc + 1) * kw])
            cos = jnp.dot(xn, wn_am[:, kc * kw:(kc + 1) * kw],
                          preferred_element_type=f32)
            cos3 = cos.reshape(spc, m_utts, kw)
            cls3 = lax.broadcasted_iota(jnp.int32, (spc, m_utts, kw),
                                        2) + kc * kw
            mask3 = cls3 == lloc
            se_k = jnp.sum(jnp.exp2(_K_AM * cos3), axis=-1, keepdims=True)
            tc_k = jnp.sum(jnp.where(mask3, cos3, 0.0), axis=-1, keepdims=True)
            mx_k = jnp.max(jnp.where(mask3, _NEG, cos3), axis=-1,
                           keepdims=True)
            se = se_k if se is None else se + se_k
            tc = tc_k if tc is None else tc + tc_k
            mxc = mx_k if mxc is None else jnp.maximum(mxc, mx_k)
        r = slice(t * ch, (t + 1) * ch)
        out_ref[0, r, 0:1] = se.reshape(ch, 1)
        out_ref[0, r, 1:2] = tc.reshape(ch, 1)
        out_ref[0, r, 2:3] = mxc.reshape(ch, 1)

    cp_px.wait()
    wn_px[...] = _colnorm_bf16(wpxf[...])

    pxc = jnp.dot(pn_ref[...], wn_px[...], preferred_element_type=f32)
    clsx = lax.broadcasted_iota(jnp.int32, (b, hc), 1)
    xmask = clsx == (labh_ref[...] - col0)
    out_ref[0, 0:b, 5:6] = jnp.sum(jnp.exp2(_K_PR * pxc), axis=-1,
                                   keepdims=True)
    out_ref[0, 0:b, 6:7] = jnp.sum(jnp.where(xmask, pxc, 0.0), axis=-1,
                                   keepdims=True)


def _combine_kernel(st_ref, out_ref, *, n, b):
    a0 = st_ref[0]
    a1 = st_ref[1]

    se_raw = a0[:, 0:1] + a1[:, 0:1]
    tc = a0[:, 1:2] + a1[:, 1:2]
    mxc = jnp.maximum(a0[:, 2:3], a1[:, 2:3])
    e_t = jnp.exp2(_K_AM * tc)
    se = se_raw + e_t * (_EXP_NEG_SM - 1.0)
    lse = jnp.log(se)
    tgt = AM_SCALE * tc - _SM
    ce_sum = jnp.sum(lse - tgt)
    hits = jnp.sum(jnp.where(tgt >= AM_SCALE * mxc, 1.0, 0.0))

    sp = a0[0:b, 3:4] + a1[0:b, 3:4]
    tp = a0[0:b, 4:5] + a1[0:b, 4:5]
    pair_sum = jnp.sum((jnp.log(sp) + PROTO_B) - (PROTO_W * tp + PROTO_B))

    sx = a0[0:b, 5:6] + a1[0:b, 5:6]
    tx = a0[0:b, 6:7] + a1[0:b, 6:7]
    proxy_sum = jnp.sum((jnp.log(sx) + PROTO_B) - (PROTO_W * tx + PROTO_B))

    loss_ce = ce_sum / float(n)
    prec1 = 100.0 * hits / float(n)
    loss_ml = 0.5 * (pair_sum / float(b)) + 0.5 * (proxy_sum / float(b))
    out_ref[0, 0] = (1.0 - MTL_WEIGHT) * loss_ce + MTL_WEIGHT * loss_ml
    out_ref[0, 1] = prec1


def kernel(x, label, w_am, w_proxy):
    B, M, D = x.shape
    C = w_am.shape[1]
    assert M >= 2
    N = B * M
    HC = C // 2
    HB = B // 2
    assert D % 128 == 0 and HC % 128 == 0 and HB % 8 == 0 and N % 8 == 0
    CH = 512 if (N % 512 == 0 and 512 % M == 0) else N
    assert CH % M == 0
    KC_N = 4 if HC % (4 * 128) == 0 else 1

    f32 = jnp.float32
    x = x.astype(f32)
    spk = label.astype(jnp.int32).reshape(B, 1)

    cost = pl.CostEstimate(
        flops=2 * N * D * C + 2 * B * D * C + 2 * B * B * D,
        transcendentals=N * C + B * C + B * B,
        bytes_accessed=2 * D * C * 4 + N * D * 4 + 2 * N * 8 * 4)

    main = functools.partial(_main_kernel, n=N, b=B, m_utts=M, hb=HB, hc=HC,
                             ch=CH, kc_n=KC_N)
    stats = pl.pallas_call(
        main,
        out_shape=jax.ShapeDtypeStruct((2, N, 8), f32),
        grid=(2,),
        in_specs=[
            pl.BlockSpec((B, M, D), lambda s: (0, 0, 0)),
            pl.BlockSpec((B, 1), lambda s: (0, 0)),
            pl.BlockSpec(memory_space=pl.ANY),
            pl.BlockSpec(memory_space=pl.ANY),
        ],
        out_specs=pl.BlockSpec((1, N, 8), lambda s: (s, 0, 0)),
        scratch_shapes=[
            pltpu.VMEM((D, HC), f32),
            pltpu.VMEM((D, HC), f32),
            pltpu.VMEM((D, HC), jnp.bfloat16),
            pltpu.VMEM((D, HC), jnp.bfloat16),
            pltpu.VMEM((B, D), jnp.bfloat16),
            pltpu.VMEM((HB, D), jnp.bfloat16),
            pltpu.SemaphoreType.DMA((KC_N,)),
            pltpu.SemaphoreType.DMA,
        ],
        compiler_params=pltpu.CompilerParams(
            dimension_semantics=("parallel",),
            vmem_limit_bytes=56 * 1024 * 1024),
        cost_estimate=cost,
    )(x, spk, w_am.astype(f32), w_proxy.astype(f32))

    comb = functools.partial(_combine_kernel, n=N, b=B)
    res = pl.pallas_call(
        comb,
        out_shape=jax.ShapeDtypeStruct((1, 2), f32),
        grid=(1,),
        in_specs=[pl.BlockSpec((2, N, 8), lambda i: (0, 0, 0))],
        out_specs=pl.BlockSpec(memory_space=pltpu.MemorySpace.SMEM),
        compiler_params=pltpu.CompilerParams(
            dimension_semantics=("arbitrary",)),
    )(stats)

    return res[0, 0], res[0, 1]


# speedup vs baseline: 1.9707x; 1.9707x over previous
import functools
import math

import jax
import jax.numpy as jnp
from jax import lax
from jax.experimental import pallas as pl
from jax.experimental.pallas import tpu as pltpu

AM_MARGIN = 0.2
AM_SCALE = 30.0
PROTO_W = 10.0
PROTO_B = -5.0
MTL_WEIGHT = 0.6

_SM = AM_SCALE * AM_MARGIN
_EXP_NEG_SM = math.exp(-_SM)
_LOG2E = 1.4426950408889634
_K_AM = AM_SCALE * _LOG2E
_K_PR = PROTO_W * _LOG2E
_NEG = -1e30


def _l2n_bf16(v):
    s = jnp.sum(v * v, axis=-1, keepdims=True)
    return (v * lax.rsqrt(jnp.maximum(s, 1e-24))).astype(jnp.bfloat16)


def _colnorm_bf16(w):
    inv = lax.rsqrt(jnp.maximum(jnp.sum(w * w, axis=0, keepdims=True), 1e-24))
    return (w * inv).astype(jnp.bfloat16)


def _main_kernel(x_ref, labr_ref, labh_ref,
                 w_am_hbm, w_px_hbm,
                 out_ref,
                 wamf, wpxf, wn_am, wn_px, pn_ref, an_ref,
                 sems_am, sem_px,
                 *, n, b, m_utts, hb, hc, ch, kc_n):
    f32 = jnp.float32
    s = pl.program_id(0)
    col0 = pl.multiple_of(s * hc, 256)
    kw = hc // kc_n

    cps_am = []
    for kc in range(kc_n):
        cp = pltpu.make_async_copy(
            w_am_hbm.at[:, pl.ds(pl.multiple_of(col0 + kc * kw, 128), kw)],
            wamf.at[:, kc * kw:(kc + 1) * kw], sems_am.at[kc])
        cp.start()
        cps_am.append(cp)
    cp_px = pltpu.make_async_copy(w_px_hbm.at[:, pl.ds(col0, hc)], wpxf,
                                  sem_px)
    cp_px.start()

    pn_ref[...] = _l2n_bf16(x_ref[:, 0, :])
    arow0 = pl.multiple_of(s * hb, 8)
    anc = x_ref[pl.ds(arow0, hb), 1, :]
    for m in range(2, m_utts):
        anc = anc + x_ref[pl.ds(arow0, hb), m, :]
    if m_utts > 2:
        anc = anc * (1.0 / float(m_utts - 1))
    an_ref[...] = _l2n_bf16(anc)

    pairc = lax.dot_general(pn_ref[...], an_ref[...], (((1,), (1,)), ((), ())),
                            preferred_element_type=f32)
    ri = lax.broadcasted_iota(jnp.int32, (b, hb), 0)
    ci = lax.broadcasted_iota(jnp.int32, (b, hb), 1) + s * hb
    pmask = ri == ci
    out_ref[0, 0:b, 3:4] = jnp.sum(jnp.exp2(_K_PR * pairc), axis=-1,
                                   keepdims=True)
    out_ref[0, 0:b, 4:5] = jnp.sum(jnp.where(pmask, pairc, 0.0), axis=-1,
                                   keepdims=True)

    spc = ch // m_utts
    for t in range(n // ch):
        xn = _l2n_bf16(x_ref[t * spc:(t + 1) * spc, :, :].reshape(ch, -1))
        lloc = labr_ref[t * ch:(t + 1) * ch, :] - col0
        se = tc = mxc = None
        for kc in range(kc_n):
            if t == 0:
                cps_am[kc].wait()
                wn_am[:, kc * kw:(kc + 1) * kw] = _colnorm_bf16(
                    wamf[:, kc * kw:(kc + 1) * kw])
            cos = jnp.dot(xn, wn_am[:, kc * kw:(kc + 1) * kw],
                          preferred_element_type=f32)
            cls = lax.broadcasted_iota(jnp.int32, (ch, kw), 1) + kc * kw
            mask = cls == lloc
            se_k = jnp.sum(jnp.exp2(_K_AM * cos), axis=-1, keepdims=True)
            tc_k = jnp.sum(jnp.where(mask, cos, 0.0), axis=-1, keepdims=True)
            mx_k = jnp.max(jnp.where(mask, _NEG, cos), axis=-1, keepdims=True)
            se = se_k if se is None else se + se_k
            tc = tc_k if tc is None else tc + tc_k
            mxc = mx_k if mxc is None else jnp.maximum(mxc, mx_k)
        r = slice(t * ch, (t + 1) * ch)
        out_ref[0, r, 0:1] = se
        out_ref[0, r, 1:2] = tc
        out_ref[0, r, 2:3] = mxc

    cp_px.wait()
    wn_px[...] = _colnorm_bf16(wpxf[...])

    pxc = jnp.dot(pn_ref[...], wn_px[...], preferred_element_type=f32)
    clsx = lax.broadcasted_iota(jnp.int32, (b, hc), 1)
    xmask = clsx == (labh_ref[...] - col0)
    out_ref[0, 0:b, 5:6] = jnp.sum(jnp.exp2(_K_PR * pxc), axis=-1,
                                   keepdims=True)
    out_ref[0, 0:b, 6:7] = jnp.sum(jnp.where(xmask, pxc, 0.0), axis=-1,
                                   keepdims=True)


def _combine_kernel(st_ref, out_ref, *, n, b):
    a0 = st_ref[0]
    a1 = st_ref[1]

    se_raw = a0[:, 0:1] + a1[:, 0:1]
    tc = a0[:, 1:2] + a1[:, 1:2]
    mxc = jnp.maximum(a0[:, 2:3], a1[:, 2:3])
    e_t = jnp.exp2(_K_AM * tc)
    se = se_raw + e_t * (_EXP_NEG_SM - 1.0)
    lse = jnp.log(se)
    tgt = AM_SCALE * tc - _SM
    ce_sum = jnp.sum(lse - tgt)
    hits = jnp.sum(jnp.where(tgt >= AM_SCALE * mxc, 1.0, 0.0))

    sp = a0[0:b, 3:4] + a1[0:b, 3:4]
    tp = a0[0:b, 4:5] + a1[0:b, 4:5]
    pair_sum = jnp.sum((jnp.log(sp) + PROTO_B) - (PROTO_W * tp + PROTO_B))

    sx = a0[0:b, 5:6] + a1[0:b, 5:6]
    tx = a0[0:b, 6:7] + a1[0:b, 6:7]
    proxy_sum = jnp.sum((jnp.log(sx) + PROTO_B) - (PROTO_W * tx + PROTO_B))

    loss_ce = ce_sum / float(n)
    prec1 = 100.0 * hits / float(n)
    loss_ml = 0.5 * (pair_sum / float(b)) + 0.5 * (proxy_sum / float(b))
    out_ref[0, 0] = (1.0 - MTL_WEIGHT) * loss_ce + MTL_WEIGHT * loss_ml
    out_ref[0, 1] = prec1


def kernel(x, label, w_am, w_proxy):
    B, M, D = x.shape
    C = w_am.shape[1]
    assert M >= 2
    N = B * M
    HC = C // 2
    HB = B // 2
    assert D % 128 == 0 and HC % 128 == 0 and HB % 8 == 0 and N % 8 == 0
    CH = 512 if (N % 512 == 0 and 512 % M == 0) else N
    assert CH % M == 0
    KC_N = 4 if HC % (4 * 128) == 0 else 1

    f32 = jnp.float32
    x = x.astype(f32)
    lab_rep = jnp.repeat(label.astype(jnp.int32), M).reshape(N, 1)
    spk = label.astype(jnp.int32).reshape(B, 1)

    cost = pl.CostEstimate(
        flops=2 * N * D * C + 2 * B * D * C + 2 * B * B * D,
        transcendentals=N * C + B * C + B * B,
        bytes_accessed=2 * D * C * 4 + N * D * 4 + 2 * N * 8 * 4)

    main = functools.partial(_main_kernel, n=N, b=B, m_utts=M, hb=HB, hc=HC,
                             ch=CH, kc_n=KC_N)
    stats = pl.pallas_call(
        main,
        out_shape=jax.ShapeDtypeStruct((2, N, 8), f32),
        grid=(2,),
        in_specs=[
            pl.BlockSpec((B, M, D), lambda s: (0, 0, 0)),
            pl.BlockSpec((N, 1), lambda s: (0, 0)),
            pl.BlockSpec((B, 1), lambda s: (0, 0)),
            pl.BlockSpec(memory_space=pl.ANY),
            pl.BlockSpec(memory_space=pl.ANY),
        ],
        out_specs=pl.BlockSpec((1, N, 8), lambda s: (s, 0, 0)),
        scratch_shapes=[
            pltpu.VMEM((D, HC), f32),
            pltpu.VMEM((D, HC), f32),
            pltpu.VMEM((D, HC), jnp.bfloat16),
            pltpu.VMEM((D, HC), jnp.bfloat16),
            pltpu.VMEM((B, D), jnp.bfloat16),
            pltpu.VMEM((HB, D), jnp.bfloat16),
            pltpu.SemaphoreType.DMA((KC_N,)),
            pltpu.SemaphoreType.DMA,
        ],
        compiler_params=pltpu.CompilerParams(
            dimension_semantics=("parallel",),
            vmem_limit_bytes=56 * 1024 * 1024),
        cost_estimate=cost,
    )(x, lab_rep, spk, w_am.astype(f32), w_proxy.astype(f32))

    comb = functools.partial(_combine_kernel, n=N, b=B)
    res = pl.pallas_call(
        comb,
        out_shape=jax.ShapeDtypeStruct((1, 2), f32),
        grid=(1,),
        in_specs=[pl.BlockSpec((2, N, 8), lambda i: (0, 0, 0))],
        out_specs=pl.BlockSpec(memory_space=pltpu.MemorySpace.SMEM),
        compiler_params=pltpu.CompilerParams(
            dimension_semantics=("arbitrary",)),
    )(stats)

    return res[0, 0], res[0, 1]


# speedup vs baseline: 2.0374x; 1.0338x over previous
import functools
import math

import jax
import jax.numpy as jnp
from jax import lax
from jax.experimental import pallas as pl
from jax.experimental.pallas import tpu as pltpu

AM_MARGIN = 0.2
AM_SCALE = 30.0
PROTO_W = 10.0
PROTO_B = -5.0
MTL_WEIGHT = 0.6

_SM = AM_SCALE * AM_MARGIN
_EXP_NEG_SM = math.exp(-_SM)
_LOG2E = 1.4426950408889634
_K_AM = AM_SCALE * _LOG2E
_K_PR = PROTO_W * _LOG2E
_NEG = -1e30


def _l2n_bf16(v):
    s = jnp.sum(v * v, axis=-1, keepdims=True)
    return (v * lax.rsqrt(jnp.maximum(s, 1e-24))).astype(jnp.bfloat16)


def _colnorm_bf16(w):
    inv = lax.rsqrt(jnp.maximum(jnp.sum(w * w, axis=0, keepdims=True), 1e-24))
    return (w * inv).astype(jnp.bfloat16)


def _main_kernel(x_ref, labr_ref, labh_ref,
                 w_am_hbm, w_px_hbm,
                 out_ref,
                 wamf, wpxf, wn_am, wn_px, pn_ref, an_ref,
                 sem_am, sem_px,
                 *, n, b, m_utts, hb, hc, ch):
    f32 = jnp.float32
    s = pl.program_id(0)
    col0 = pl.multiple_of(s * hc, 256)

    cp_am = pltpu.make_async_copy(w_am_hbm.at[:, pl.ds(col0, hc)], wamf, sem_am)
    cp_am.start()
    cp_px = pltpu.make_async_copy(w_px_hbm.at[:, pl.ds(col0, hc)], wpxf, sem_px)
    cp_px.start()

    pn_ref[...] = _l2n_bf16(x_ref[:, 0, :])
    arow0 = pl.multiple_of(s * hb, 8)
    anc = x_ref[pl.ds(arow0, hb), 1, :]
    for m in range(2, m_utts):
        anc = anc + x_ref[pl.ds(arow0, hb), m, :]
    if m_utts > 2:
        anc = anc * (1.0 / float(m_utts - 1))
    an_ref[...] = _l2n_bf16(anc)

    pairc = lax.dot_general(pn_ref[...], an_ref[...], (((1,), (1,)), ((), ())),
                            preferred_element_type=f32)
    ri = lax.broadcasted_iota(jnp.int32, (b, hb), 0)
    ci = lax.broadcasted_iota(jnp.int32, (b, hb), 1) + s * hb
    pmask = ri == ci
    out_ref[0, 0:b, 3:4] = jnp.sum(jnp.exp2(_K_PR * pairc), axis=-1,
                                   keepdims=True)
    out_ref[0, 0:b, 4:5] = jnp.sum(jnp.where(pmask, pairc, 0.0), axis=-1,
                                   keepdims=True)

    cp_am.wait()
    wn_am[...] = _colnorm_bf16(wamf[...])

    spc = ch // m_utts
    for t in range(n // ch):
        xn = _l2n_bf16(x_ref[t * spc:(t + 1) * spc, :, :].reshape(ch, -1))
        cos = jnp.dot(xn, wn_am[...], preferred_element_type=f32)
        cls = lax.broadcasted_iota(jnp.int32, (ch, hc), 1)
        mask = cls == (labr_ref[t * ch:(t + 1) * ch, :] - col0)
        r = slice(t * ch, (t + 1) * ch)
        out_ref[0, r, 0:1] = jnp.sum(jnp.exp2(_K_AM * cos), axis=-1,
                                     keepdims=True)
        out_ref[0, r, 1:2] = jnp.sum(jnp.where(mask, cos, 0.0), axis=-1,
                                     keepdims=True)
        out_ref[0, r, 2:3] = jnp.max(jnp.where(mask, _NEG, cos), axis=-1,
                                     keepdims=True)

    cp_px.wait()
    wn_px[...] = _colnorm_bf16(wpxf[...])

    pxc = jnp.dot(pn_ref[...], wn_px[...], preferred_element_type=f32)
    clsx = lax.broadcasted_iota(jnp.int32, (b, hc), 1)
    xmask = clsx == (labh_ref[...] - col0)
    out_ref[0, 0:b, 5:6] = jnp.sum(jnp.exp2(_K_PR * pxc), axis=-1,
                                   keepdims=True)
    out_ref[0, 0:b, 6:7] = jnp.sum(jnp.where(xmask, pxc, 0.0), axis=-1,
                                   keepdims=True)


def _combine_kernel(st_ref, out_ref, *, n, b):
    a0 = st_ref[0]
    a1 = st_ref[1]

    se_raw = a0[:, 0:1] + a1[:, 0:1]
    tc = a0[:, 1:2] + a1[:, 1:2]
    mxc = jnp.maximum(a0[:, 2:3], a1[:, 2:3])
    e_t = jnp.exp2(_K_AM * tc)
    se = se_raw + e_t * (_EXP_NEG_SM - 1.0)
    lse = jnp.log(se)
    tgt = AM_SCALE * tc - _SM
    ce_sum = jnp.sum(lse - tgt)
    hits = jnp.sum(jnp.where(tgt >= AM_SCALE * mxc, 1.0, 0.0))

    sp = a0[0:b, 3:4] + a1[0:b, 3:4]
    tp = a0[0:b, 4:5] + a1[0:b, 4:5]
    pair_sum = jnp.sum((jnp.log(sp) + PROTO_B) - (PROTO_W * tp + PROTO_B))

    sx = a0[0:b, 5:6] + a1[0:b, 5:6]
    tx = a0[0:b, 6:7] + a1[0:b, 6:7]
    proxy_sum = jnp.sum((jnp.log(sx) + PROTO_B) - (PROTO_W * tx + PROTO_B))

    loss_ce = ce_sum / float(n)
    prec1 = 100.0 * hits / float(n)
    loss_ml = 0.5 * (pair_sum / float(b)) + 0.5 * (proxy_sum / float(b))
    out_ref[0, 0] = (1.0 - MTL_WEIGHT) * loss_ce + MTL_WEIGHT * loss_ml
    out_ref[0, 1] = prec1


def kernel(x, label, w_am, w_proxy):
    B, M, D = x.shape
    C = w_am.shape[1]
    assert M >= 2
    N = B * M
    HC = C // 2
    HB = B // 2
    assert D % 128 == 0 and HC % 128 == 0 and HB % 8 == 0 and N % 8 == 0
    CH = 512 if (N % 512 == 0 and 512 % M == 0) else N
    assert CH % M == 0

    f32 = jnp.float32
    x = x.astype(f32)
    lab_rep = jnp.repeat(label.astype(jnp.int32), M).reshape(N, 1)
    spk = label.astype(jnp.int32).reshape(B, 1)

    cost = pl.CostEstimate(
        flops=2 * N * D * C + 2 * B * D * C + 2 * B * B * D,
        transcendentals=N * C + B * C + B * B,
        bytes_accessed=2 * D * C * 4 + N * D * 4 + 2 * N * 8 * 4)

    main = functools.partial(_main_kernel, n=N, b=B, m_utts=M, hb=HB, hc=HC,
                             ch=CH)
    stats = pl.pallas_call(
        main,
        out_shape=jax.ShapeDtypeStruct((2, N, 8), f32),
        grid=(2,),
        in_specs=[
            pl.BlockSpec((B, M, D), lambda s: (0, 0, 0)),
            pl.BlockSpec((N, 1), lambda s: (0, 0)),
            pl.BlockSpec((B, 1), lambda s: (0, 0)),
            pl.BlockSpec(memory_space=pl.ANY),
            pl.BlockSpec(memory_space=pl.ANY),
        ],
        out_specs=pl.BlockSpec((1, N, 8), lambda s: (s, 0, 0)),
        scratch_shapes=[
            pltpu.VMEM((D, HC), f32),
            pltpu.VMEM((D, HC), f32),
            pltpu.VMEM((D, HC), jnp.bfloat16),
            pltpu.VMEM((D, HC), jnp.bfloat16),
            pltpu.VMEM((B, D), jnp.bfloat16),
            pltpu.VMEM((HB, D), jnp.bfloat16),
            pltpu.SemaphoreType.DMA,
            pltpu.SemaphoreType.DMA,
        ],
        compiler_params=pltpu.CompilerParams(
            dimension_semantics=("parallel",),
            vmem_limit_bytes=56 * 1024 * 1024),
        cost_estimate=cost,
    )(x, lab_rep, spk, w_am.astype(f32), w_proxy.astype(f32))

    comb = functools.partial(_combine_kernel, n=N, b=B)
    res = pl.pallas_call(
        comb,
        out_shape=jax.ShapeDtypeStruct((1, 2), f32),
        grid=(1,),
        in_specs=[pl.BlockSpec((2, N, 8), lambda i: (0, 0, 0))],
        out_specs=pl.BlockSpec(memory_space=pltpu.MemorySpace.SMEM),
        compiler_params=pltpu.CompilerParams(
            dimension_semantics=("arbitrary",)),
    )(stats)

    return res[0, 0], res[0, 1]
